# Initial kernel scaffold; baseline (speedup 1.0000x reference)
import jax
import jax.numpy as jnp
from jax.experimental import pallas as pl


def kernel(x, table):
    raise NotImplementedError("write your pallas kernel here")


# speedup vs baseline: 1.2134x; 1.2134x over previous
import functools

import jax
import jax.numpy as jnp
import numpy as np
from jax import lax
from jax.experimental import pallas as pl
from jax.experimental.pallas import tpu as pltpu
from jax.experimental.pallas import tpu_sc as plsc

B = 4
S = 4096
D = 768
LANES = 16
KV = D // LANES

NC, NS = 2, 16
NW = NC * NS
POS_PER_W = S // NW
C = 16
NCH = POS_PER_W // C
ROWS = B * C


def _pe_np() -> np.ndarray:
    pos = np.arange(S, dtype=np.float32)[:, None]
    i = np.arange(0, D, 2, dtype=np.float32)
    div = np.power(10000.0, (i / np.float32(D)).astype(np.float32))
    pe = np.zeros((S, D), np.float32)
    pe[:, 0::2] = np.sin(pos / div)
    pe[:, 1::2] = np.cos(pos / div)
    return pe


_PE = _pe_np()

_MESH = plsc.VectorSubcoreMesh(core_axis_name="c", subcore_axis_name="s")


@functools.partial(
    pl.kernel,
    mesh=_MESH,
    out_type=jax.ShapeDtypeStruct((B, S, D), jnp.float32),
    scratch_types=[
        pltpu.VMEM((NCH, ROWS), jnp.int32),
        pltpu.VMEM((ROWS, D), jnp.float32),
        pltpu.VMEM((C, D), jnp.float32),
        pltpu.SemaphoreType.DMA,
    ],
)
def _emb_kernel(x_hbm, table_hbm, pe_hbm, out_hbm, idx_v, rows_v, pe_v, sem):
    wid = lax.axis_index("c") * NS + lax.axis_index("s")
    base = wid * POS_PER_W

    for j in range(NCH):
        for b in range(B):
            pltpu.sync_copy(
                x_hbm.at[b, pl.ds(base + j * C, C)],
                idx_v.at[j, pl.ds(b * C, C)],
            )

    for j in range(NCH):
        pos = base + j * C
        pltpu.async_copy(table_hbm.at[idx_v.at[j]], rows_v, sem).wait()
        pltpu.sync_copy(pe_hbm.at[pl.ds(pos, C)], pe_v)

        def _row_body(r, _):
            def _col_body(k, _):
                off = k * LANES
                p = pe_v[r, pl.ds(off, LANES)]
                for b in range(B):
                    row = b * C + r
                    rows_v[row, pl.ds(off, LANES)] = (
                        rows_v[row, pl.ds(off, LANES)] + p
                    )
                return 0

            lax.fori_loop(0, KV, _col_body, 0)
            return 0

        lax.fori_loop(0, C, _row_body, 0)

        for b in range(B):
            pltpu.sync_copy(
                rows_v.at[pl.ds(b * C, C)],
                out_hbm.at[b, pl.ds(pos, C)],
            )


def kernel(x, table):
    pe = jnp.asarray(_PE)
    return _emb_kernel(x, table, pe)


# speedup vs baseline: 3.3859x; 2.7905x over previous
import functools

import jax
import jax.numpy as jnp
import numpy as np
from jax import lax
from jax.experimental import pallas as pl
from jax.experimental.pallas import tpu as pltpu
from jax.experimental.pallas import tpu_sc as plsc

B = 4
S = 4096
D = 768
LANES = 16
KV = D // LANES
KU = 6

NC, NS = 2, 16
NW = NC * NS
POS_PER_W = S // NW
C = 16
NCH = POS_PER_W // C
ROWS = B * C


def _pe_np() -> np.ndarray:
    pos = np.arange(S, dtype=np.float32)[:, None]
    i = np.arange(0, D, 2, dtype=np.float32)
    div = np.power(10000.0, (i / np.float32(D)).astype(np.float32))
    pe = np.zeros((S, D), np.float32)
    pe[:, 0::2] = np.sin(pos / div)
    pe[:, 1::2] = np.cos(pos / div)
    return pe


_PE = _pe_np()

_MESH = plsc.VectorSubcoreMesh(core_axis_name="c", subcore_axis_name="s")


@functools.partial(
    pl.kernel,
    mesh=_MESH,
    out_type=jax.ShapeDtypeStruct((B, S, D), jnp.float32),
    scratch_types=[
        pltpu.VMEM((B, POS_PER_W), jnp.int32),
        pltpu.VMEM((ROWS, D), jnp.float32),
        pltpu.VMEM((ROWS, D), jnp.float32),
        pltpu.VMEM((C, D), jnp.float32),
        pltpu.VMEM((C, D), jnp.float32),
        pltpu.SemaphoreType.DMA,
        pltpu.SemaphoreType.DMA,
        pltpu.SemaphoreType.DMA,
        pltpu.SemaphoreType.DMA,
        pltpu.SemaphoreType.DMA,
        pltpu.SemaphoreType.DMA,
        pltpu.SemaphoreType.DMA,
    ],
)
def _emb_kernel(x_hbm, table_hbm, pe_hbm, out_hbm,
                xtmp, rows0, rows1, pe0, pe1,
                xsem, gsem0, gsem1, psem0, psem1, osem0, osem1):
    rows = (rows0, rows1)
    pes = (pe0, pe1)
    gsems = (gsem0, gsem1)
    psems = (psem0, psem1)
    osems = (osem0, osem1)

    wid = lax.axis_index("c") * NS + lax.axis_index("s")
    base = wid * POS_PER_W

    xhs = [
        pltpu.async_copy(x_hbm.at[b, pl.ds(base, POS_PER_W)], xtmp.at[b], xsem)
        for b in range(B)
    ]
    for h in xhs:
        h.wait()

    def start_chunk(j):
        buf = j % 2
        ghs = [
            pltpu.async_copy(
                table_hbm.at[xtmp.at[b, pl.ds(j * C, C)]],
                rows[buf].at[pl.ds(b * C, C)],
                gsems[buf],
            )
            for b in range(B)
        ]
        ph = pltpu.async_copy(
            pe_hbm.at[pl.ds(base + j * C, C)], pes[buf], psems[buf]
        )
        return ghs, ph

    out_hs = [None, None]
    pending = {0: start_chunk(0)}
    for j in range(NCH):
        cur = j % 2
        nxt = 1 - cur
        if j + 1 < NCH:
            if out_hs[nxt] is not None:
                for h in out_hs[nxt]:
                    h.wait()
                out_hs[nxt] = None
            pending[j + 1] = start_chunk(j + 1)
        ghs, ph = pending.pop(j)
        for h in ghs:
            h.wait()
        ph.wait()

        rbuf = rows[cur]
        pbuf = pes[cur]

        def _row_body(r, _):
            @plsc.parallel_loop(0, KV, 1, unroll=KU)
            def _col_body(k):
                off = k * LANES
                p = pbuf[r, pl.ds(off, LANES)]
                for b in range(B):
                    row = b * C + r
                    rbuf[row, pl.ds(off, LANES)] = (
                        rbuf[row, pl.ds(off, LANES)] + p
                    )

            return 0

        lax.fori_loop(0, C, _row_body, 0)

        out_hs[cur] = [
            pltpu.async_copy(
                rbuf.at[pl.ds(b * C, C)],
                out_hbm.at[b, pl.ds(base + j * C, C)],
                osems[cur],
            )
            for b in range(B)
        ]
    for hs in out_hs:
        if hs is not None:
            for h in hs:
                h.wait()


def kernel(x, table):
    pe = jnp.asarray(_PE)
    return _emb_kernel(x, table, pe)
